# Initial kernel scaffold; baseline (speedup 1.0000x reference)
import jax
import jax.numpy as jnp
from jax.experimental import pallas as pl


def kernel(video_frames, attn_in_w, attn_in_b, attn_out_w, attn_out_b, ln1_g, ln1_b, ffn_w1, ffn_b1, ffn_w2, ffn_b2, ln2_g, ln2_b):
    raise NotImplementedError("write your pallas kernel here")


# speedup vs baseline: 258.4825x; 258.4825x over previous
import functools
import math

import jax
import jax.numpy as jnp
from jax.experimental import pallas as pl
from jax.experimental.pallas import tpu as pltpu

_D = 768
_NH = 8


def _norms_body(x_ref, o_ref):
    x = x_ref[...]
    o_ref[...] = jnp.sqrt(jnp.sum(x * x, axis=1, keepdims=True))


def _row_norms(x):
    n = x.shape[0]
    return pl.pallas_call(
        _norms_body,
        out_shape=jax.ShapeDtypeStruct((n, 1), jnp.float32),
    )(x)


def _top2_body(xr_ref, xf_ref, nr_ref, nc_ref, o_ref, *, n):
    blk = xr_ref.shape[0]
    dots = jax.lax.dot_general(
        xr_ref[...], xf_ref[...], (((1,), (1,)), ((), ())),
        preferred_element_type=jnp.float32)
    denom = jnp.maximum(nr_ref[...] * nc_ref[...], 1e-8)
    sim = dots / denom
    colid = jax.lax.broadcasted_iota(jnp.int32, (blk, n), 1)
    m1 = jnp.max(sim, axis=1, keepdims=True)
    i1 = jnp.min(jnp.where(sim == m1, colid, n), axis=1, keepdims=True)
    sim2 = jnp.where(colid == i1, -jnp.inf, sim)
    m2 = jnp.max(sim2, axis=1, keepdims=True)
    i2 = jnp.min(jnp.where(sim2 == m2, colid, n), axis=1, keepdims=True)
    lane = jax.lax.broadcasted_iota(jnp.int32, (blk, 128), 1)
    o_ref[...] = jnp.where(lane == 0, i1, jnp.where(lane == 1, i2, 0))


def _top2_pairs(x, norms, k):
    n = x.shape[0]
    blk = 200 if k % 400 else 400
    grid = k // blk
    nc = norms.reshape(1, n)
    out = pl.pallas_call(
        functools.partial(_top2_body, n=n),
        grid=(grid,),
        in_specs=[
            pl.BlockSpec((blk, _D), lambda i: (i, 0)),
            pl.BlockSpec((n, _D), lambda i: (0, 0)),
            pl.BlockSpec((blk, 1), lambda i: (i, 0)),
            pl.BlockSpec((1, n), lambda i: (0, 0)),
        ],
        out_specs=pl.BlockSpec((blk, 128), lambda i: (i, 0)),
        out_shape=jax.ShapeDtypeStruct((k, 128), jnp.int32),
    )(x, x, norms, nc)
    return out[:, :2]


def _scan_body(pairs_ref, x_ref, o_ref, buf_ref, *, k, keep, wa, wb):
    buf_ref[...] = x_ref[...]

    def step(j, carry):
        a = pairs_ref[j, 0]
        b = pairs_ref[j, 1]
        ra = buf_ref[pl.ds(a, 1), :]
        rb = buf_ref[pl.ds(b, 1), :]
        new = ra * wa + rb * wb
        buf_ref[pl.ds(a, 1), :] = new
        buf_ref[pl.ds(b, 1), :] = new
        return carry

    jax.lax.fori_loop(0, k, step, 0, unroll=False)
    o_ref[...] = buf_ref[pl.ds(0, keep), :]


def _merge_scan(x, pairs, k, wa, wb):
    n = x.shape[0]
    keep = n - k
    return pl.pallas_call(
        functools.partial(_scan_body, k=k, keep=keep, wa=wa, wb=wb),
        in_specs=[
            pl.BlockSpec(memory_space=pltpu.SMEM),
            pl.BlockSpec(memory_space=pltpu.VMEM),
        ],
        out_specs=pl.BlockSpec(memory_space=pltpu.VMEM),
        out_shape=jax.ShapeDtypeStruct((keep, _D), jnp.float32),
        scratch_shapes=[pltpu.VMEM((n, _D), jnp.float32)],
    )(pairs, x)


def _merge(tokens, ratio, wa, wb):
    n = tokens.shape[0]
    k = int(n * ratio)
    norms = _row_norms(tokens)
    pairs = _top2_pairs(tokens, norms, k)
    return _merge_scan(tokens, pairs, k, wa, wb)


def _qkv_body(x_ref, w_ref, b_ref, o_ref):
    o_ref[...] = jax.lax.dot_general(
        x_ref[...], w_ref[...], (((1,), (1,)), ((), ())),
        preferred_element_type=jnp.float32) + b_ref[...]


def _attn_body(q_ref, k_ref, v_ref, o_ref, *, scale):
    q = q_ref[0]
    k = k_ref[0]
    v = v_ref[0]
    s = jax.lax.dot_general(
        q, k, (((1,), (1,)), ((), ())),
        preferred_element_type=jnp.float32) / scale
    m = jnp.max(s, axis=1, keepdims=True)
    e = jnp.exp(s - m)
    p = e / jnp.sum(e, axis=1, keepdims=True)
    o_ref[0] = jax.lax.dot_general(
        p, v, (((1,), (0,)), ((), ())), preferred_element_type=jnp.float32)


def _ln(r, g_ref, b_ref):
    m = jnp.mean(r, axis=1, keepdims=True)
    v = jnp.mean((r - m) ** 2, axis=1, keepdims=True)
    return (r - m) / jnp.sqrt(v + 1e-5) * g_ref[...] + b_ref[...]


def _proj_ln_body(o_ref, w_ref, b_ref, x_ref, g_ref, bb_ref, out_ref):
    y = jax.lax.dot_general(
        o_ref[...], w_ref[...], (((1,), (1,)), ((), ())),
        preferred_element_type=jnp.float32) + b_ref[...]
    out_ref[...] = _ln(x_ref[...] + y, g_ref, bb_ref)


def _mha_block(x, in_w, in_b, out_w, out_b, g, b):
    l = x.shape[0]
    hd = _D // _NH
    qkv = pl.pallas_call(
        _qkv_body,
        out_shape=jax.ShapeDtypeStruct((l, 3 * _D), jnp.float32),
    )(x, in_w, in_b.reshape(1, -1))
    q, k, v = jnp.split(qkv, 3, axis=-1)
    q = q.reshape(l, _NH, hd).transpose(1, 0, 2)
    k = k.reshape(l, _NH, hd).transpose(1, 0, 2)
    v = v.reshape(l, _NH, hd).transpose(1, 0, 2)
    o = pl.pallas_call(
        functools.partial(_attn_body, scale=hd ** 0.5),
        grid=(_NH,),
        in_specs=[
            pl.BlockSpec((1, l, hd), lambda i: (i, 0, 0)),
            pl.BlockSpec((1, l, hd), lambda i: (i, 0, 0)),
            pl.BlockSpec((1, l, hd), lambda i: (i, 0, 0)),
        ],
        out_specs=pl.BlockSpec((1, l, hd), lambda i: (i, 0, 0)),
        out_shape=jax.ShapeDtypeStruct((_NH, l, hd), jnp.float32),
    )(q, k, v)
    o = o.transpose(1, 0, 2).reshape(l, _D)
    return pl.pallas_call(
        _proj_ln_body,
        out_shape=jax.ShapeDtypeStruct((l, _D), jnp.float32),
    )(o, out_w, out_b.reshape(1, -1), x, g.reshape(1, -1), b.reshape(1, -1))


def _ffn1_body(x_ref, w_ref, b_ref, o_ref):
    h = jax.lax.dot_general(
        x_ref[...], w_ref[...], (((1,), (1,)), ((), ())),
        preferred_element_type=jnp.float32) + b_ref[...]
    o_ref[...] = 0.5 * h * (1.0 + jax.lax.erf(h * math.sqrt(0.5)))


def _ffn2_body(h_ref, w_ref, b_ref, x_ref, g_ref, bb_ref, o_ref):
    y = jax.lax.dot_general(
        h_ref[...], w_ref[...], (((1,), (1,)), ((), ())),
        preferred_element_type=jnp.float32) + b_ref[...]
    o_ref[...] = _ln(x_ref[...] + y, g_ref, bb_ref)


def _ffn_block(x, w1, b1, w2, b2, g, b):
    l = x.shape[0]
    dh = w1.shape[0]
    nblk = 4
    h = pl.pallas_call(
        _ffn1_body,
        grid=(nblk,),
        in_specs=[
            pl.BlockSpec((l, _D), lambda i: (0, 0)),
            pl.BlockSpec((dh // nblk, _D), lambda i: (i, 0)),
            pl.BlockSpec((1, dh // nblk), lambda i: (0, i)),
        ],
        out_specs=pl.BlockSpec((l, dh // nblk), lambda i: (0, i)),
        out_shape=jax.ShapeDtypeStruct((l, dh), jnp.float32),
    )(x, w1, b1.reshape(1, -1))
    return pl.pallas_call(
        _ffn2_body,
        out_shape=jax.ShapeDtypeStruct((l, _D), jnp.float32),
    )(h, w2, b2.reshape(1, -1), x, g.reshape(1, -1), b.reshape(1, -1))


def kernel(video_frames, attn_in_w, attn_in_b, attn_out_w, attn_out_b,
           ln1_g, ln1_b, ffn_w1, ffn_b1, ffn_w2, ffn_b2, ln2_g, ln2_b):
    tokens = video_frames.reshape(-1, video_frames.shape[-1])
    tokens = _merge(tokens, 0.5, 0.5, 0.5)
    tokens = _merge(tokens, 0.5, 0.5, 0.5)
    tokens = _mha_block(tokens, attn_in_w, attn_in_b, attn_out_w, attn_out_b,
                        ln1_g, ln1_b)
    tokens = _merge(tokens, 0.25, 0.6, 0.4)
    return _ffn_block(tokens, ffn_w1, ffn_b1, ffn_w2, ffn_b2, ln2_g, ln2_b)
